# baseline (speedup 1.0000x reference)
import functools

import jax
import jax.numpy as jnp
from jax import lax
from jax.experimental import pallas as pl
from jax.experimental.pallas import tpu as pltpu
from jax.experimental.pallas import tpu_sc as plsc

B, M, N = 16, 4, 128
H, P = 256, 32
G = B * M
NLOOK = G * N
CHUNK = 128
H2 = H // 2
GPB = 8
GPM = GPB // M
GC = G // 2
B2 = B // 2
NL2 = NLOOK // 2


def _sc_gather(tab_a, tab_f, idx_a, idx_f, nlook):
    info = plsc.get_sparse_core_info()
    nc, ns = info.num_cores, info.num_subcores
    nw = nc * ns
    bpw = nlook // nw
    nchunk = bpw // CHUNK

    mesh = plsc.VectorSubcoreMesh(core_axis_name="c", subcore_axis_name="s")

    @functools.partial(
        pl.kernel,
        out_type=jax.ShapeDtypeStruct((2, nlook, H2), jnp.int32),
        mesh=mesh,
        scratch_types=[
            pltpu.VMEM((2, nchunk, CHUNK), jnp.int32),
            pltpu.VMEM((2, bpw, H2), jnp.int32),
            pltpu.SemaphoreType.DMA,
            pltpu.SemaphoreType.DMA,
        ],
    )
    def gather_kernel(a_tab, f_tab, a_idx, f_idx, out, idx_v, rows_v,
                      sem_i, sem_g):
        wid = lax.axis_index("s") * nc + lax.axis_index("c")
        base = wid * bpw
        ic0 = pltpu.async_copy(a_idx.at[wid], idx_v.at[0], sem_i)
        ic1 = pltpu.async_copy(f_idx.at[wid], idx_v.at[1], sem_i)
        ic0.wait()
        ic1.wait()
        copies = []
        for t, tab in ((0, a_tab), (1, f_tab)):
            for j in range(nchunk):
                copies.append(pltpu.async_copy(
                    tab.at[idx_v.at[t, j]],
                    rows_v.at[t, pl.ds(j * CHUNK, CHUNK)], sem_g))
        for c in copies:
            c.wait()
        pltpu.sync_copy(rows_v, out.at[:, pl.ds(base, bpw)])

    return gather_kernel(tab_a, tab_f, idx_a, idx_f)


def _graph_block(g, adj_ref, xa_ref, xf_ref, wa_ref, wf_ref, bi_ref,
                 w1_ref, b1_ref, w2_ref, b2_ref, ratios_ref, roff):
    f32 = jnp.float32
    bf16 = jnp.bfloat16

    def unpack(ref):
        w = ref[...].reshape(GPB * N, H2)
        lo = jax.lax.bitcast_convert_type(w << 16, f32)
        hi = jax.lax.bitcast_convert_type(w & jnp.int32(-65536), f32)
        return jnp.concatenate([lo, hi], axis=1).astype(bf16)

    xa = unpack(xa_ref)
    xf = unpack(xf_ref)
    x = (jnp.dot(xa, wa_ref[...], preferred_element_type=f32)
         + jnp.dot(xf, wf_ref[...], preferred_element_type=f32)
         + bi_ref[...])
    x = jnp.maximum(x, 0.0)

    a3 = adj_ref[...].astype(f32)
    r = lax.broadcasted_iota(jnp.int32, (GPB, N, N), 1)
    c = lax.broadcasted_iota(jnp.int32, (GPB, N, N), 2)
    a3 = a3 + (r == c).astype(f32)
    deg = jnp.sum(a3, axis=2)
    dinv = (1.0 / jnp.sqrt(deg)).reshape(GPB * N, 1)
    a3b = a3.astype(bf16)

    def gcn_layer(x, w_ref, b_ref):
        t = dinv * jnp.dot(x.astype(bf16), w_ref[...],
                           preferred_element_type=f32)
        tb = t.astype(bf16)
        parts = [jnp.dot(a3b[i], tb[i * N:(i + 1) * N],
                         preferred_element_type=f32) for i in range(GPB)]
        s = jnp.concatenate(parts, axis=0)
        return jnp.maximum(dinv * s + b_ref[...], 0.0)

    x = gcn_layer(x, w1_ref, b1_ref)
    x = gcn_layer(x, w2_ref, b2_ref)
    mol = jnp.mean(x.reshape(GPB, N, H), axis=1)

    r_rows = ratios_ref[pl.ds(roff + g * GPM, GPM), :]
    w_rows = r_rows / (jnp.sum(r_rows, axis=1, keepdims=True) + 1e-8)
    return jnp.sum(w_rows[:, :, None] * mol.reshape(GPM, M, H), axis=1)


def _half1_body(adj_ref, xa_ref, xf_ref, wa_ref, wf_ref, bi_ref,
                w1_ref, b1_ref, w2_ref, b2_ref, ratios_ref, mix_ref):
    g = pl.program_id(0)
    mix = _graph_block(g, adj_ref, xa_ref, xf_ref, wa_ref, wf_ref, bi_ref,
                       w1_ref, b1_ref, w2_ref, b2_ref, ratios_ref, 0)
    mix_ref[...] = mix.reshape(1, GPM, H)


def _half2_body(adj_ref, xa_ref, xf_ref, wa_ref, wf_ref, bi_ref,
                w1_ref, b1_ref, w2_ref, b2_ref, ratios_ref,
                phys_ref, gh_ref, gp_ref, eh_ref, ep_ref,
                wh1h_ref, wh1p_ref, bh1_ref, wh2_ref, bh2_ref, part1_ref,
                y_ref, mix_s):
    f32 = jnp.float32
    g = pl.program_id(0)
    mix = _graph_block(g, adj_ref, xa_ref, xf_ref, wa_ref, wf_ref, bi_ref,
                       w1_ref, b1_ref, w2_ref, b2_ref, ratios_ref, B2)
    mix_s[pl.ds(g, 1)] = mix.reshape(1, GPM, H)

    @pl.when(g == B2 // GPM - 1)
    def head():
        ratios = ratios_ref[...]
        w = ratios / (jnp.sum(ratios, axis=1, keepdims=True) + 1e-8)
        phys = phys_ref[...]
        phys = jnp.where(jnp.isnan(phys), 0.0, phys)
        phys = jnp.clip(phys, -1000.0, 1000.0)
        mix_p = jnp.sum(w[:, :, None] * phys, axis=1)
        mix_h = jnp.concatenate(
            [part1_ref[...].reshape(B2, H), mix_s[...].reshape(B2, H)],
            axis=0)

        hp = float(H + P)
        mu = (jnp.sum(mix_h, axis=1, keepdims=True)
              + jnp.sum(mix_p, axis=1, keepdims=True)) / hp
        dh = mix_h - mu
        dp = mix_p - mu
        var = (jnp.sum(dh * dh, axis=1, keepdims=True)
               + jnp.sum(dp * dp, axis=1, keepdims=True)) / hp
        inv = 1.0 / jnp.sqrt(var + 1e-5)
        znh = dh * inv * gh_ref[...] + eh_ref[...]
        znp = dp * inv * gp_ref[...] + ep_ref[...]

        h = jnp.maximum(znh @ wh1h_ref[...] + znp @ wh1p_ref[...]
                        + bh1_ref[...], 0.0)
        y = h @ wh2_ref[...] + bh2_ref[...]
        big = jnp.finfo(f32).max
        y_ref[...] = jnp.where(jnp.isnan(y), 0.0, jnp.clip(y, -big, big))


_REP2 = pl.BlockSpec((H, H), lambda g: (0, 0))
_REPB = pl.BlockSpec((1, H), lambda g: (0, 0))
_COMMON_SPECS = [
    pl.BlockSpec((GPB, N, N), lambda g: (g, 0, 0)),
    pl.BlockSpec((1, GPB, N, H2), lambda g: (0, g, 0, 0)),
    pl.BlockSpec((1, GPB, N, H2), lambda g: (1, g, 0, 0)),
    _REP2, _REP2, _REPB, _REP2, _REPB, _REP2, _REPB,
    pl.BlockSpec((B, M), lambda g: (0, 0)),
]


def _half1_forward(adj, xa, xf, *ws):
    return pl.pallas_call(
        _half1_body,
        grid=(GC // GPB,),
        in_specs=list(_COMMON_SPECS),
        out_specs=pl.BlockSpec((1, GPM, H), lambda g: (g, 0, 0)),
        out_shape=jax.ShapeDtypeStruct((B2 // GPM, GPM, H), jnp.float32),
    )(adj, xa, xf, *ws)


def _half2_forward(adj, xa, xf, *ws):
    return pl.pallas_call(
        _half2_body,
        grid=(GC // GPB,),
        in_specs=list(_COMMON_SPECS) + [
            pl.BlockSpec((B, M, P), lambda g: (0, 0, 0)),
            _REPB, pl.BlockSpec((1, P), lambda g: (0, 0)),
            _REPB, pl.BlockSpec((1, P), lambda g: (0, 0)),
            _REP2, pl.BlockSpec((P, H), lambda g: (0, 0)),
            _REPB,
            pl.BlockSpec((H, 1), lambda g: (0, 0)),
            pl.BlockSpec((1, 1), lambda g: (0, 0)),
            pl.BlockSpec((B2 // GPM, GPM, H), lambda g: (0, 0, 0)),
        ],
        out_specs=pl.BlockSpec((B, 1), lambda g: (0, 0)),
        out_shape=jax.ShapeDtypeStruct((B, 1), jnp.float32),
        scratch_shapes=[pltpu.VMEM((B2 // GPM, GPM, H), jnp.float32)],
    )(adj, xa, xf, *ws)


def _pack_rows(t):
    tb = t.astype(jnp.bfloat16)
    lo = jax.lax.bitcast_convert_type(tb[:, :H2], jnp.uint16).astype(jnp.uint32)
    hi = jax.lax.bitcast_convert_type(tb[:, H2:], jnp.uint16).astype(jnp.uint32)
    return jax.lax.bitcast_convert_type((hi << 16) | lo, jnp.int32)


def kernel(af, fp, adj, phys, ratios, atom_emb, fp_emb, W_in, b_in,
           W1, b1, W2, b2, ln_g, ln_b, Wh1, bh1, Wh2, bh2):
    info = plsc.get_sparse_core_info()
    nw = info.num_cores * info.num_subcores
    ishape = (nw, NL2 // nw // CHUNK, CHUNK)
    af32 = af.astype(jnp.int32).reshape(2, NL2)
    fp32 = fp.astype(jnp.int32).reshape(2, NL2)
    ta, tf = _pack_rows(atom_emb), _pack_rows(fp_emb)

    packed1 = _sc_gather(ta, tf, af32[0].reshape(ishape),
                         fp32[0].reshape(ishape), NL2)
    packed2 = _sc_gather(ta, tf, af32[1].reshape(ishape),
                         fp32[1].reshape(ishape), NL2)

    bf16 = jnp.bfloat16
    adj8 = adj.reshape(2, GC, N, N).astype(jnp.int8)
    ws = (W_in[:H].astype(bf16), W_in[H:].astype(bf16), b_in.reshape(1, H),
          W1.astype(bf16), b1.reshape(1, H), W2.astype(bf16),
          b2.reshape(1, H), ratios)
    part1 = _half1_forward(adj8[0], packed1.reshape(2, GC, N, H2),
                           packed1.reshape(2, GC, N, H2), *ws)
    y = _half2_forward(
        adj8[1], packed2.reshape(2, GC, N, H2),
        packed2.reshape(2, GC, N, H2), *ws,
        phys,
        ln_g[:H].reshape(1, H), ln_g[H:].reshape(1, P),
        ln_b[:H].reshape(1, H), ln_b[H:].reshape(1, P),
        Wh1[:H], Wh1[H:], bh1.reshape(1, H),
        Wh2, bh2.reshape(1, 1),
        part1,
    )
    return y


# speedup vs baseline: 1.0632x; 1.0632x over previous
import functools

import jax
import jax.numpy as jnp
from jax import lax
from jax.experimental import pallas as pl
from jax.experimental.pallas import tpu as pltpu
from jax.experimental.pallas import tpu_sc as plsc

B, M, N = 16, 4, 128
H, P = 256, 32
G = B * M
NLOOK = G * N
CHUNK = 128
H2 = H // 2
GPB = 8
GPM = GPB // M


def _sc_gather(tab_a, tab_f, idx_a, idx_f):
    info = plsc.get_sparse_core_info()
    nc, ns = info.num_cores, info.num_subcores
    nw = nc * ns
    bpw = NLOOK // nw
    nchunk = bpw // CHUNK

    mesh = plsc.VectorSubcoreMesh(core_axis_name="c", subcore_axis_name="s")

    @functools.partial(
        pl.kernel,
        out_type=jax.ShapeDtypeStruct((2, NLOOK, H2), jnp.int32),
        mesh=mesh,
        scratch_types=[
            pltpu.VMEM((2, nchunk, CHUNK), jnp.int32),
            pltpu.VMEM((2, bpw, H2), jnp.int32),
            pltpu.SemaphoreType.DMA,
            pltpu.SemaphoreType.DMA,
        ],
    )
    def gather_kernel(a_tab, f_tab, a_idx, f_idx, out, idx_v, rows_v,
                      sem_i, sem_g):
        wid = lax.axis_index("s") * nc + lax.axis_index("c")
        base = wid * bpw
        ic0 = pltpu.async_copy(a_idx.at[wid], idx_v.at[0], sem_i)
        ic1 = pltpu.async_copy(f_idx.at[wid], idx_v.at[1], sem_i)
        ic0.wait()
        ic1.wait()
        copies = []
        for t, tab in ((0, a_tab), (1, f_tab)):
            for j in range(nchunk):
                copies.append(pltpu.async_copy(
                    tab.at[idx_v.at[t, j]],
                    rows_v.at[t, pl.ds(j * CHUNK, CHUNK)], sem_g))
        for c in copies:
            c.wait()
        pltpu.sync_copy(rows_v, out.at[:, pl.ds(base, bpw)])

    return gather_kernel(tab_a, tab_f, idx_a, idx_f)


def _fused_body(adj_ref, xa_ref, xf_ref, wa_ref, wf_ref, bi_ref,
                w1_ref, b1_ref, w2_ref, b2_ref,
                ratios_ref, phys_ref, gh_ref, gp_ref, eh_ref, ep_ref,
                wh1h_ref, wh1p_ref, bh1_ref, wh2_ref, bh2_ref,
                y_ref, mixh_ref):
    f32 = jnp.float32
    bf16 = jnp.bfloat16
    g = pl.program_id(0)

    def unpack(ref):
        w = ref[...].reshape(GPB * N, H2)
        lo = jax.lax.bitcast_convert_type(w << 16, f32)
        hi = jax.lax.bitcast_convert_type(w & jnp.int32(-65536), f32)
        return jnp.concatenate([lo, hi], axis=1).astype(bf16)

    xa = unpack(xa_ref)
    xf = unpack(xf_ref)
    x = (jnp.dot(xa, wa_ref[...], preferred_element_type=f32)
         + jnp.dot(xf, wf_ref[...], preferred_element_type=f32)
         + bi_ref[...])
    x = jnp.maximum(x, 0.0)

    a3 = adj_ref[...].astype(f32)
    r = lax.broadcasted_iota(jnp.int32, (GPB, N, N), 1)
    c = lax.broadcasted_iota(jnp.int32, (GPB, N, N), 2)
    a3 = a3 + (r == c).astype(f32)
    deg = jnp.sum(a3, axis=2)
    dinv = (1.0 / jnp.sqrt(deg)).reshape(GPB * N, 1)
    a3b = a3.astype(bf16)

    def gcn_layer(x, w_ref, b_ref):
        t = dinv * jnp.dot(x.astype(bf16), w_ref[...],
                           preferred_element_type=f32)
        tb = t.astype(bf16)
        parts = [jnp.dot(a3b[i], tb[i * N:(i + 1) * N],
                         preferred_element_type=f32) for i in range(GPB)]
        s = jnp.concatenate(parts, axis=0)
        return jnp.maximum(dinv * s + b_ref[...], 0.0)

    x = gcn_layer(x, w1_ref, b1_ref)
    x = gcn_layer(x, w2_ref, b2_ref)
    mol = jnp.mean(x.reshape(GPB, N, H), axis=1)

    r_rows = ratios_ref[pl.ds(g * GPM, GPM), :]
    w_rows = r_rows / (jnp.sum(r_rows, axis=1, keepdims=True) + 1e-8)
    mix = jnp.sum(w_rows[:, :, None] * mol.reshape(GPM, M, H), axis=1)
    mixh_ref[pl.ds(g, 1)] = mix.reshape(1, GPM, H)

    @pl.when(g == B // GPM - 1)
    def head():
        ratios = ratios_ref[...]
        w = ratios / (jnp.sum(ratios, axis=1, keepdims=True) + 1e-8)
        phys = phys_ref[...]
        phys = jnp.where(jnp.isnan(phys), 0.0, phys)
        phys = jnp.clip(phys, -1000.0, 1000.0)
        mix_p = jnp.sum(w[:, :, None] * phys, axis=1)
        mix_h = mixh_ref[...].reshape(B, H)

        hp = float(H + P)
        mu = (jnp.sum(mix_h, axis=1, keepdims=True)
              + jnp.sum(mix_p, axis=1, keepdims=True)) / hp
        dh = mix_h - mu
        dp = mix_p - mu
        var = (jnp.sum(dh * dh, axis=1, keepdims=True)
               + jnp.sum(dp * dp, axis=1, keepdims=True)) / hp
        inv = 1.0 / jnp.sqrt(var + 1e-5)
        znh = dh * inv * gh_ref[...] + eh_ref[...]
        znp = dp * inv * gp_ref[...] + ep_ref[...]

        h = jnp.maximum(znh @ wh1h_ref[...] + znp @ wh1p_ref[...]
                        + bh1_ref[...], 0.0)
        y = h @ wh2_ref[...] + bh2_ref[...]
        big = jnp.finfo(f32).max
        y_ref[...] = jnp.where(jnp.isnan(y), 0.0, jnp.clip(y, -big, big))


def _fused_forward(adj, xa, xf, w_in_a, w_in_f, b_in, w1, b1, w2, b2,
                   ratios, phys, gh, gp, eh, ep, wh1h, wh1p, bh1, wh2, bh2):
    rep2 = pl.BlockSpec((H, H), lambda g: (0, 0))
    repb = pl.BlockSpec((1, H), lambda g: (0, 0))
    return pl.pallas_call(
        _fused_body,
        grid=(G // GPB,),
        in_specs=[
            pl.BlockSpec((GPB, N, N), lambda g: (g, 0, 0)),
            pl.BlockSpec((1, GPB, N, H2), lambda g: (0, g, 0, 0)),
            pl.BlockSpec((1, GPB, N, H2), lambda g: (1, g, 0, 0)),
            rep2, rep2, repb, rep2, repb, rep2, repb,
            pl.BlockSpec((B, M), lambda g: (0, 0)),
            pl.BlockSpec((B, M, P), lambda g: (0, 0, 0)),
            repb, pl.BlockSpec((1, P), lambda g: (0, 0)),
            repb, pl.BlockSpec((1, P), lambda g: (0, 0)),
            rep2, pl.BlockSpec((P, H), lambda g: (0, 0)),
            repb,
            pl.BlockSpec((H, 1), lambda g: (0, 0)),
            pl.BlockSpec((1, 1), lambda g: (0, 0)),
        ],
        out_specs=pl.BlockSpec((B, 1), lambda g: (0, 0)),
        out_shape=jax.ShapeDtypeStruct((B, 1), jnp.float32),
        scratch_shapes=[pltpu.VMEM((B // GPM, GPM, H), jnp.float32)],
    )(adj, xa, xf, w_in_a, w_in_f, b_in, w1, b1, w2, b2,
      ratios, phys, gh, gp, eh, ep, wh1h, wh1p, bh1, wh2, bh2)


def _pack_rows(t):
    tb = t.astype(jnp.bfloat16)
    lo = jax.lax.bitcast_convert_type(tb[:, :H2], jnp.uint16).astype(jnp.uint32)
    hi = jax.lax.bitcast_convert_type(tb[:, H2:], jnp.uint16).astype(jnp.uint32)
    return jax.lax.bitcast_convert_type((hi << 16) | lo, jnp.int32)


def kernel(af, fp, adj, phys, ratios, atom_emb, fp_emb, W_in, b_in,
           W1, b1, W2, b2, ln_g, ln_b, Wh1, bh1, Wh2, bh2):
    info = plsc.get_sparse_core_info()
    nw = info.num_cores * info.num_subcores
    ishape = (nw, NLOOK // nw // CHUNK, CHUNK)
    packed = _sc_gather(_pack_rows(atom_emb), _pack_rows(fp_emb),
                        af.astype(jnp.int32).reshape(ishape),
                        fp.astype(jnp.int32).reshape(ishape))

    bf16 = jnp.bfloat16
    packed4 = packed.reshape(2, G, N, H2)
    y = _fused_forward(
        adj.reshape(G, N, N).astype(jnp.int8), packed4, packed4,
        W_in[:H].astype(bf16), W_in[H:].astype(bf16),
        b_in.reshape(1, H),
        W1.astype(bf16), b1.reshape(1, H), W2.astype(bf16),
        b2.reshape(1, H),
        ratios, phys,
        ln_g[:H].reshape(1, H), ln_g[H:].reshape(1, P),
        ln_b[:H].reshape(1, H), ln_b[H:].reshape(1, P),
        Wh1[:H], Wh1[H:], bh1.reshape(1, H),
        Wh2, bh2.reshape(1, 1),
    )
    return y


# speedup vs baseline: 1.0744x; 1.0106x over previous
import functools

import jax
import jax.numpy as jnp
from jax import lax
from jax.experimental import pallas as pl
from jax.experimental.pallas import tpu as pltpu
from jax.experimental.pallas import tpu_sc as plsc

B, M, N = 16, 4, 128
H, P = 256, 32
G = B * M
NLOOK = G * N
CHUNK = 128
H2 = H // 2
GPB = 16
GPM = GPB // M


def _sc_gather(tab_a, tab_f, idx_a, idx_f):
    info = plsc.get_sparse_core_info()
    nc, ns = info.num_cores, info.num_subcores
    nw = nc * ns
    bpw = NLOOK // nw
    nchunk = bpw // CHUNK

    mesh = plsc.VectorSubcoreMesh(core_axis_name="c", subcore_axis_name="s")

    @functools.partial(
        pl.kernel,
        out_type=jax.ShapeDtypeStruct((2, NLOOK, H2), jnp.int32),
        mesh=mesh,
        scratch_types=[
            pltpu.VMEM((2, nchunk, CHUNK), jnp.int32),
            pltpu.VMEM((2, bpw, H2), jnp.int32),
            pltpu.SemaphoreType.DMA,
            pltpu.SemaphoreType.DMA,
        ],
    )
    def gather_kernel(a_tab, f_tab, a_idx, f_idx, out, idx_v, rows_v,
                      sem_i, sem_g):
        wid = lax.axis_index("s") * nc + lax.axis_index("c")
        base = wid * bpw
        ic0 = pltpu.async_copy(a_idx.at[wid], idx_v.at[0], sem_i)
        ic1 = pltpu.async_copy(f_idx.at[wid], idx_v.at[1], sem_i)
        ic0.wait()
        ic1.wait()
        copies = []
        for t, tab in ((0, a_tab), (1, f_tab)):
            for j in range(nchunk):
                copies.append(pltpu.async_copy(
                    tab.at[idx_v.at[t, j]],
                    rows_v.at[t, pl.ds(j * CHUNK, CHUNK)], sem_g))
        for c in copies:
            c.wait()
        pltpu.sync_copy(rows_v, out.at[:, pl.ds(base, bpw)])

    return gather_kernel(tab_a, tab_f, idx_a, idx_f)


def _fused_body(adj_ref, xa_ref, xf_ref, wa_ref, wf_ref, bi_ref,
                w1_ref, b1_ref, w2_ref, b2_ref,
                ratios_ref, phys_ref, gh_ref, gp_ref, eh_ref, ep_ref,
                wh1h_ref, wh1p_ref, bh1_ref, wh2_ref, bh2_ref,
                y_ref, mixh_ref):
    f32 = jnp.float32
    bf16 = jnp.bfloat16
    g = pl.program_id(0)

    def unpack(ref):
        w = ref[...].reshape(GPB * N, H2)
        lo = jax.lax.bitcast_convert_type(w << 16, f32)
        hi = jax.lax.bitcast_convert_type(w & jnp.int32(-65536), f32)
        return jnp.concatenate([lo, hi], axis=1).astype(bf16)

    xa = unpack(xa_ref)
    xf = unpack(xf_ref)
    x = (jnp.dot(xa, wa_ref[...], preferred_element_type=f32)
         + jnp.dot(xf, wf_ref[...], preferred_element_type=f32)
         + bi_ref[...])
    x = jnp.maximum(x, 0.0)

    a3 = adj_ref[...].astype(f32)
    r = lax.broadcasted_iota(jnp.int32, (GPB, N, N), 1)
    c = lax.broadcasted_iota(jnp.int32, (GPB, N, N), 2)
    a3 = a3 + (r == c).astype(f32)
    deg = jnp.sum(a3, axis=2)
    dinv = (1.0 / jnp.sqrt(deg)).reshape(GPB * N, 1)
    a3b = a3.astype(bf16)

    def gcn_layer(x, w_ref, b_ref):
        t = dinv * jnp.dot(x.astype(bf16), w_ref[...],
                           preferred_element_type=f32)
        tb = t.astype(bf16)
        parts = [jnp.dot(a3b[i], tb[i * N:(i + 1) * N],
                         preferred_element_type=f32) for i in range(GPB)]
        s = jnp.concatenate(parts, axis=0)
        return jnp.maximum(dinv * s + b_ref[...], 0.0)

    x = gcn_layer(x, w1_ref, b1_ref)
    x = gcn_layer(x, w2_ref, b2_ref)
    mol = jnp.mean(x.reshape(GPB, N, H), axis=1)

    r_rows = ratios_ref[pl.ds(g * GPM, GPM), :]
    w_rows = r_rows / (jnp.sum(r_rows, axis=1, keepdims=True) + 1e-8)
    mix = jnp.sum(w_rows[:, :, None] * mol.reshape(GPM, M, H), axis=1)
    mixh_ref[pl.ds(g, 1)] = mix.reshape(1, GPM, H)

    @pl.when(g == B // GPM - 1)
    def head():
        ratios = ratios_ref[...]
        w = ratios / (jnp.sum(ratios, axis=1, keepdims=True) + 1e-8)
        phys = phys_ref[...]
        phys = jnp.where(jnp.isnan(phys), 0.0, phys)
        phys = jnp.clip(phys, -1000.0, 1000.0)
        mix_p = jnp.sum(w[:, :, None] * phys, axis=1)
        mix_h = mixh_ref[...].reshape(B, H)

        hp = float(H + P)
        mu = (jnp.sum(mix_h, axis=1, keepdims=True)
              + jnp.sum(mix_p, axis=1, keepdims=True)) / hp
        dh = mix_h - mu
        dp = mix_p - mu
        var = (jnp.sum(dh * dh, axis=1, keepdims=True)
               + jnp.sum(dp * dp, axis=1, keepdims=True)) / hp
        inv = 1.0 / jnp.sqrt(var + 1e-5)
        znh = dh * inv * gh_ref[...] + eh_ref[...]
        znp = dp * inv * gp_ref[...] + ep_ref[...]

        h = jnp.maximum(znh @ wh1h_ref[...] + znp @ wh1p_ref[...]
                        + bh1_ref[...], 0.0)
        y = h @ wh2_ref[...] + bh2_ref[...]
        big = jnp.finfo(f32).max
        y_ref[...] = jnp.where(jnp.isnan(y), 0.0, jnp.clip(y, -big, big))


def _fused_forward(adj, xa, xf, w_in_a, w_in_f, b_in, w1, b1, w2, b2,
                   ratios, phys, gh, gp, eh, ep, wh1h, wh1p, bh1, wh2, bh2):
    rep2 = pl.BlockSpec((H, H), lambda g: (0, 0))
    repb = pl.BlockSpec((1, H), lambda g: (0, 0))
    return pl.pallas_call(
        _fused_body,
        grid=(G // GPB,),
        in_specs=[
            pl.BlockSpec((GPB, N, N), lambda g: (g, 0, 0)),
            pl.BlockSpec((1, GPB, N, H2), lambda g: (0, g, 0, 0)),
            pl.BlockSpec((1, GPB, N, H2), lambda g: (1, g, 0, 0)),
            rep2, rep2, repb, rep2, repb, rep2, repb,
            pl.BlockSpec((B, M), lambda g: (0, 0)),
            pl.BlockSpec((B, M, P), lambda g: (0, 0, 0)),
            repb, pl.BlockSpec((1, P), lambda g: (0, 0)),
            repb, pl.BlockSpec((1, P), lambda g: (0, 0)),
            rep2, pl.BlockSpec((P, H), lambda g: (0, 0)),
            repb,
            pl.BlockSpec((H, 1), lambda g: (0, 0)),
            pl.BlockSpec((1, 1), lambda g: (0, 0)),
        ],
        out_specs=pl.BlockSpec((B, 1), lambda g: (0, 0)),
        out_shape=jax.ShapeDtypeStruct((B, 1), jnp.float32),
        scratch_shapes=[pltpu.VMEM((B // GPM, GPM, H), jnp.float32)],
    )(adj, xa, xf, w_in_a, w_in_f, b_in, w1, b1, w2, b2,
      ratios, phys, gh, gp, eh, ep, wh1h, wh1p, bh1, wh2, bh2)


def _pack_rows(t):
    tb = t.astype(jnp.bfloat16)
    lo = jax.lax.bitcast_convert_type(tb[:, :H2], jnp.uint16).astype(jnp.uint32)
    hi = jax.lax.bitcast_convert_type(tb[:, H2:], jnp.uint16).astype(jnp.uint32)
    return jax.lax.bitcast_convert_type((hi << 16) | lo, jnp.int32)


def kernel(af, fp, adj, phys, ratios, atom_emb, fp_emb, W_in, b_in,
           W1, b1, W2, b2, ln_g, ln_b, Wh1, bh1, Wh2, bh2):
    info = plsc.get_sparse_core_info()
    nw = info.num_cores * info.num_subcores
    ishape = (nw, NLOOK // nw // CHUNK, CHUNK)
    packed = _sc_gather(_pack_rows(atom_emb), _pack_rows(fp_emb),
                        af.astype(jnp.int32).reshape(ishape),
                        fp.astype(jnp.int32).reshape(ishape))

    bf16 = jnp.bfloat16
    packed4 = packed.reshape(2, G, N, H2)
    y = _fused_forward(
        adj.reshape(G, N, N).astype(jnp.int8), packed4, packed4,
        W_in[:H].astype(bf16), W_in[H:].astype(bf16),
        b_in.reshape(1, H),
        W1.astype(bf16), b1.reshape(1, H), W2.astype(bf16),
        b2.reshape(1, H),
        ratios, phys,
        ln_g[:H].reshape(1, H), ln_g[H:].reshape(1, P),
        ln_b[:H].reshape(1, H), ln_b[H:].reshape(1, P),
        Wh1[:H], Wh1[H:], bh1.reshape(1, H),
        Wh2, bh2.reshape(1, 1),
    )
    return y
